# baseline (speedup 1.0000x reference)
import functools

import jax
import jax.numpy as jnp
from jax.experimental import pallas as pl
from jax.experimental.pallas import tpu as pltpu

_G = 64


def _body(x_ref, bt_ref, w1_ref, b1_ref, w2_ref, out_ref, acc_ref, m_ref, z_ref,
          *, n_rows, block_r, n_blocks):
    i = pl.program_id(0)

    @pl.when(i == 0)
    def _init():
        acc_ref[...] = jnp.zeros_like(acc_ref)
        m_ref[0, 0] = -1e30
        z_ref[0, 0] = 0.0

    x_blk = x_ref[...]
    h_t = jax.lax.dot_general(w1_ref[...], x_blk,
                              (((0,), (1,)), ((), ())),
                              preferred_element_type=jnp.float32)
    h_t = jnp.maximum(h_t + b1_ref[...], 0.0)
    logits = jnp.sum(h_t * w2_ref[...], axis=0, keepdims=True)

    if n_rows != block_r * n_blocks:
        col = i * block_r + jax.lax.broadcasted_iota(jnp.int32, (1, block_r), 1)
        logits = jnp.where(col < n_rows, logits, -1e30)

    m_old = m_ref[0, 0]
    m_new = jnp.maximum(m_old, jnp.max(logits))
    alpha = jnp.exp(m_old - m_new)
    w = jnp.exp(logits - m_new)
    z_ref[0, 0] = z_ref[0, 0] * alpha + jnp.sum(w)
    m_ref[0, 0] = m_new

    ids = jax.lax.broadcasted_iota(jnp.int32, (_G, block_r), 0)
    onehot_w = jnp.where(ids == bt_ref[0], w, 0.0)
    contrib = jnp.dot(onehot_w, x_blk, preferred_element_type=jnp.float32)
    acc_ref[...] = acc_ref[...] * alpha + contrib

    @pl.when(i == n_blocks - 1)
    def _fin():
        out_ref[...] = acc_ref[...] / z_ref[0, 0]


def kernel(x, batch, W1, b1, W2, b2):
    n, d = x.shape
    h_dim = W1.shape[1]
    block_r = 5000
    n_blocks = -(-n // block_r)
    n_pad = n_blocks * block_r

    x_p = x if n_pad == n else jnp.pad(x, ((0, n_pad - n), (0, 0)))
    bt = batch.astype(jnp.int32)
    if n_pad != n:
        bt = jnp.pad(bt, (0, n_pad - n), constant_values=-1)
    bt3 = bt.reshape(n_blocks, 1, block_r)
    b1c = b1.reshape(h_dim, 1).astype(jnp.float32)
    w2c = W2.reshape(h_dim, 1).astype(jnp.float32)

    body = functools.partial(_body, n_rows=n, block_r=block_r, n_blocks=n_blocks)
    out = pl.pallas_call(
        body,
        grid=(n_blocks,),
        in_specs=[
            pl.BlockSpec((block_r, d), lambda i: (i, 0)),
            pl.BlockSpec((1, 1, block_r), lambda i: (i, 0, 0)),
            pl.BlockSpec((d, h_dim), lambda i: (0, 0)),
            pl.BlockSpec((h_dim, 1), lambda i: (0, 0)),
            pl.BlockSpec((h_dim, 1), lambda i: (0, 0)),
        ],
        out_specs=pl.BlockSpec((_G, d), lambda i: (0, 0)),
        out_shape=jax.ShapeDtypeStruct((_G, d), jnp.float32),
        scratch_shapes=[
            pltpu.VMEM((_G, d), jnp.float32),
            pltpu.SMEM((1, 1), jnp.float32),
            pltpu.SMEM((1, 1), jnp.float32),
        ],
    )(x_p, bt3, W1, b1c, w2c)
    return out


# speedup vs baseline: 1.0396x; 1.0396x over previous
import functools

import jax
import jax.numpy as jnp
from jax.experimental import pallas as pl
from jax.experimental.pallas import tpu as pltpu

_G = 64


def _body(x_ref, bt_ref, w1_ref, b1_ref, w2_ref, out_ref, acc_ref, m_ref, z_ref,
          *, n_rows, block_r, n_blocks):
    i = pl.program_id(0)

    @pl.when(i == 0)
    def _init():
        acc_ref[...] = jnp.zeros_like(acc_ref)
        m_ref[0, 0] = -1e30
        z_ref[0, 0] = 0.0

    x_blk = x_ref[...]
    h_t = jax.lax.dot_general(w1_ref[...], x_blk,
                              (((0,), (1,)), ((), ())),
                              preferred_element_type=jnp.float32)
    h_t = jnp.maximum(h_t + b1_ref[...], 0.0)
    logits = jnp.sum(h_t * w2_ref[...], axis=0, keepdims=True)

    if n_rows != block_r * n_blocks:
        col = i * block_r + jax.lax.broadcasted_iota(jnp.int32, (1, block_r), 1)
        logits = jnp.where(col < n_rows, logits, -1e30)

    m_old = m_ref[0, 0]
    m_new = jnp.maximum(m_old, jnp.max(logits))
    alpha = jnp.exp(m_old - m_new)
    w = jnp.exp(logits - m_new)
    z_ref[0, 0] = z_ref[0, 0] * alpha + jnp.sum(w)
    m_ref[0, 0] = m_new

    ids = jax.lax.broadcasted_iota(jnp.int32, (_G, block_r), 0)
    onehot_w = jnp.where(ids == bt_ref[0], w, 0.0)
    contrib = jnp.dot(onehot_w, x_blk, preferred_element_type=jnp.float32)
    acc_ref[...] = acc_ref[...] * alpha + contrib

    @pl.when(i == n_blocks - 1)
    def _fin():
        out_ref[...] = acc_ref[...] / z_ref[0, 0]


def kernel(x, batch, W1, b1, W2, b2):
    n, d = x.shape
    h_dim = W1.shape[1]
    block_r = 10000
    n_blocks = -(-n // block_r)
    n_pad = n_blocks * block_r

    x_p = x if n_pad == n else jnp.pad(x, ((0, n_pad - n), (0, 0)))
    bt = batch.astype(jnp.int32)
    if n_pad != n:
        bt = jnp.pad(bt, (0, n_pad - n), constant_values=-1)
    bt3 = bt.reshape(n_blocks, 1, block_r)
    b1c = b1.reshape(h_dim, 1).astype(jnp.float32)
    w2c = W2.reshape(h_dim, 1).astype(jnp.float32)

    body = functools.partial(_body, n_rows=n, block_r=block_r, n_blocks=n_blocks)
    out = pl.pallas_call(
        body,
        grid=(n_blocks,),
        in_specs=[
            pl.BlockSpec((block_r, d), lambda i: (i, 0)),
            pl.BlockSpec((1, 1, block_r), lambda i: (i, 0, 0)),
            pl.BlockSpec((d, h_dim), lambda i: (0, 0)),
            pl.BlockSpec((h_dim, 1), lambda i: (0, 0)),
            pl.BlockSpec((h_dim, 1), lambda i: (0, 0)),
        ],
        out_specs=pl.BlockSpec((_G, d), lambda i: (0, 0)),
        out_shape=jax.ShapeDtypeStruct((_G, d), jnp.float32),
        scratch_shapes=[
            pltpu.VMEM((_G, d), jnp.float32),
            pltpu.SMEM((1, 1), jnp.float32),
            pltpu.SMEM((1, 1), jnp.float32),
        ],
    )(x_p, bt3, W1, b1c, w2c)
    return out
